# baseline (speedup 1.0000x reference)
import functools

import jax
import jax.numpy as jnp
from jax import lax
from jax.experimental import pallas as pl
from jax.experimental.pallas import tpu as pltpu
from jax.experimental.pallas import tpu_sc as plsc

N_VOCAB = 49408
N_EMBD = 768
N_TOKENS = 77
BATCH = 1024

NC = 2
NS = 16
NW = NC * NS
SEQ_W = BATCH // NW
QD = 256
NPASS = N_EMBD // QD
NBUF = 4
LANES = 16
G = QD // LANES
RU = 7
PT = 80


def _embed_body(idx_hbm, table_hbm, pos_hbm, out_hbm,
                idx_v, pos_v, buf0, buf1, buf2, buf3,
                gs0, gs1, gs2, gs3, ss0, ss1, ss2, ss3):
    bufs = (buf0, buf1, buf2, buf3)
    gsems = (gs0, gs1, gs2, gs3)
    ssems = (ss0, ss1, ss2, ss3)

    cid = lax.axis_index("c")
    sid = lax.axis_index("s")
    wid = sid * NC + cid
    seq0 = wid * SEQ_W

    pltpu.sync_copy(idx_hbm.at[wid], idx_v)

    for p in range(NPASS):
        dsl = pl.ds(p * QD, QD)
        pltpu.sync_copy(pos_hbm.at[:, dsl], pos_v)

        def issue_gather(s, b):
            pltpu.async_copy(table_hbm.at[idx_v.at[s], dsl], bufs[b], gsems[b])

        def wait_gather(b):
            pltpu.make_async_copy(
                table_hbm.at[idx_v.at[0], dsl], bufs[b], gsems[b]).wait()

        def issue_store(s, b):
            pltpu.async_copy(
                bufs[b], out_hbm.at[seq0 + s, :, dsl], ssems[b])

        def wait_store(b):
            pltpu.make_async_copy(
                bufs[b], out_hbm.at[0, :, dsl], ssems[b]).wait()

        def add_pos(b):
            def row_body(to, _):
                for r in range(RU):
                    t = to * RU + r
                    for g in range(G):
                        sl = pl.ds(g * LANES, LANES)
                        plsc.addupdate(bufs[b].at[t, sl], pos_v[t, sl])
                return 0
            lax.fori_loop(0, N_TOKENS // RU, row_body, 0, unroll=False)

        issue_gather(0, 0)
        issue_gather(1, 1)

        def outer(so, _):
            for b in range(NBUF):
                s = so * NBUF + b
                wait_gather(b)
                add_pos(b)
                issue_store(s, b)
                k = s + 2
                bk = (b + 2) % NBUF
                @pl.when(s >= 2)
                def _():
                    wait_store(bk)

                @pl.when(k < SEQ_W)
                def _():
                    issue_gather(k, bk)
            return 0

        lax.fori_loop(0, SEQ_W // NBUF, outer, 0, unroll=False)

        wait_store((SEQ_W - 2) % NBUF)
        wait_store((SEQ_W - 1) % NBUF)


@functools.partial(
    pl.kernel,
    out_type=jax.ShapeDtypeStruct((BATCH, PT, N_EMBD), jnp.float32),
    mesh=plsc.VectorSubcoreMesh(core_axis_name="c", subcore_axis_name="s"),
    scratch_types=[
        pltpu.VMEM((SEQ_W, PT), jnp.int32),
        pltpu.VMEM((N_TOKENS, QD), jnp.float32),
        pltpu.VMEM((PT, QD), jnp.float32),
        pltpu.VMEM((PT, QD), jnp.float32),
        pltpu.VMEM((PT, QD), jnp.float32),
        pltpu.VMEM((PT, QD), jnp.float32),
        pltpu.SemaphoreType.DMA,
        pltpu.SemaphoreType.DMA,
        pltpu.SemaphoreType.DMA,
        pltpu.SemaphoreType.DMA,
        pltpu.SemaphoreType.DMA,
        pltpu.SemaphoreType.DMA,
        pltpu.SemaphoreType.DMA,
        pltpu.SemaphoreType.DMA,
    ],
)
def _embed_kernel(idx_hbm, table_hbm, pos_hbm, out_hbm, *scratch):
    _embed_body(idx_hbm, table_hbm, pos_hbm, out_hbm, *scratch)


def kernel(tokens, token_embedding, position_embedding):
    idx = jnp.pad(jnp.asarray(tokens, jnp.int32),
                  ((0, 0), (0, PT - N_TOKENS))).reshape(NW, SEQ_W, PT)
    out = _embed_kernel(idx, token_embedding, position_embedding)
    return out[:, :N_TOKENS, :]


# speedup vs baseline: 1.9220x; 1.9220x over previous
import functools

import jax
import jax.numpy as jnp
from jax import lax
from jax.experimental import pallas as pl
from jax.experimental.pallas import tpu as pltpu
from jax.experimental.pallas import tpu_sc as plsc

N_VOCAB = 49408
N_EMBD = 768
N_TOKENS = 77
BATCH = 1024

NC = 2
NS = 16
NW = NC * NS
SEQ_W = BATCH // NW
LANES = 16
G = N_EMBD // LANES
RU = 7


def _embed_body(idx_hbm, table_hbm, pos_hbm, out_hbm,
                idx_v, pos_v, buf, gsem, ssem):
    cid = lax.axis_index("c")
    sid = lax.axis_index("s")
    wid = sid * NC + cid
    seq0 = wid * SEQ_W

    pltpu.sync_copy(idx_hbm.at[wid], idx_v)
    pltpu.sync_copy(pos_hbm, pos_v)

    def seq_loop(s, _):
        pltpu.async_copy(table_hbm.at[idx_v.at[s]], buf, gsem).wait()

        def row_body(to, _):
            for r in range(RU):
                t = to * RU + r
                for g in range(G):
                    sl = pl.ds(g * LANES, LANES)
                    plsc.addupdate(buf.at[t, sl], pos_v[t, sl])
            return 0
        lax.fori_loop(0, N_TOKENS // RU, row_body, 0, unroll=False)

        pltpu.async_copy(buf, out_hbm.at[seq0 + s], ssem).wait()
        return 0

    lax.fori_loop(0, SEQ_W, seq_loop, 0, unroll=False)


@functools.partial(
    pl.kernel,
    out_type=jax.ShapeDtypeStruct((BATCH, N_TOKENS, N_EMBD), jnp.float32),
    mesh=plsc.VectorSubcoreMesh(core_axis_name="c", subcore_axis_name="s"),
    scratch_types=[
        pltpu.VMEM((SEQ_W, N_TOKENS), jnp.int32),
        pltpu.VMEM((N_TOKENS, N_EMBD), jnp.float32),
        pltpu.VMEM((N_TOKENS, N_EMBD), jnp.float32),
        pltpu.SemaphoreType.DMA,
        pltpu.SemaphoreType.DMA,
    ],
)
def _embed_kernel(idx_hbm, table_hbm, pos_hbm, out_hbm, *scratch):
    _embed_body(idx_hbm, table_hbm, pos_hbm, out_hbm, *scratch)


def kernel(tokens, token_embedding, position_embedding):
    idx = jnp.asarray(tokens, jnp.int32).reshape(NW, SEQ_W, N_TOKENS)
    return _embed_kernel(idx, token_embedding, position_embedding)
